# baseline (speedup 1.0000x reference)
import functools

import jax
import jax.numpy as jnp
from jax import lax
from jax.experimental import pallas as pl
from jax.experimental.pallas import tpu as pltpu
from jax.experimental.pallas import tpu_sc as plsc

B = 4096
L = 50
N = B * L
G_E = 15
G_O = 16
VOCAB = 1000000
NC, NS = 2, 16
NW = NC * NS
ROWS_PER_W = N // NW
CHUNK = 128
N_CHUNKS = ROWS_PER_W // CHUNK

_mesh = plsc.VectorSubcoreMesh(core_axis_name="c", subcore_axis_name="s")


@functools.partial(
    pl.kernel,
    mesh=_mesh,
    out_type=jax.ShapeDtypeStruct((N, G_O, 4), jnp.float32),
    scratch_types=[
        pltpu.VMEM((CHUNK,), jnp.int32),
        pltpu.VMEM((CHUNK, G_E, 4), jnp.float32),
        pltpu.VMEM((CHUNK, 1, 4), jnp.float32),
        pltpu.SemaphoreType.DMA,
    ],
    compiler_params=pltpu.CompilerParams(use_tc_tiling_on_sc=False),
)
def _emb_kernel(x_hbm, coords_hbm, table_hbm, out_hbm, idx_v, rows_v, crd_v, sem):
    wid = lax.axis_index("s") * NC + lax.axis_index("c")
    base0 = wid * ROWS_PER_W

    def body(i, carry):
        base = base0 + i * CHUNK
        pltpu.sync_copy(x_hbm.at[pl.ds(base, CHUNK)], idx_v)
        pltpu.async_copy(table_hbm.at[idx_v], rows_v, sem).wait()
        pltpu.sync_copy(coords_hbm.at[pl.ds(base, CHUNK)], crd_v)
        pltpu.sync_copy(rows_v, out_hbm.at[pl.ds(base, CHUNK), pl.ds(0, G_E)])
        pltpu.sync_copy(crd_v, out_hbm.at[pl.ds(base, CHUNK), pl.ds(G_E, 1)])
        return carry

    lax.fori_loop(0, N_CHUNKS, body, 0)


def kernel(x, coordinates, table):
    xf = x.reshape(N).astype(jnp.int32)
    cf = coordinates.reshape(N, 1, 4)
    tf = table.reshape(VOCAB, G_E, 4)
    out = _emb_kernel(xf, cf, tf)
    return out.reshape(B, L, 64)


# speedup vs baseline: 21.2790x; 21.2790x over previous
import functools

import jax
import jax.numpy as jnp
from jax import lax
from jax.experimental import pallas as pl
from jax.experimental.pallas import tpu as pltpu
from jax.experimental.pallas import tpu_sc as plsc

B = 4096
L = 50
D_O = 64
VOCAB = 1000000
NC, NS = 2, 16
NW = NC * NS
B_PER_W = B // NW
NB = 8
N_CHUNKS = B_PER_W // NB

_mesh = plsc.VectorSubcoreMesh(core_axis_name="c", subcore_axis_name="s")


@functools.partial(
    pl.kernel,
    mesh=_mesh,
    out_type=jax.ShapeDtypeStruct((B, L, D_O), jnp.float32),
    scratch_types=[
        pltpu.VMEM((NB, L), jnp.int32),
        pltpu.VMEM((NB, L, 16), jnp.float32),
        pltpu.VMEM((NB, L, D_O), jnp.float32),
        pltpu.SemaphoreType.DMA,
    ],
    compiler_params=pltpu.CompilerParams(use_tc_tiling_on_sc=False),
)
def _emb_kernel(x_hbm, coords_hbm, table_hbm, out_hbm, xv, cv, obuf, sem):
    wid = lax.axis_index("s") * NC + lax.axis_index("c")
    b0 = wid * B_PER_W

    lane = lax.iota(jnp.int32, 16)
    is_emb = lane < 12

    def chunk_body(i, carry):
        b = b0 + i * NB
        pltpu.sync_copy(x_hbm.at[pl.ds(b, NB)], xv)
        copies = [
            pltpu.async_copy(table_hbm.at[xv.at[j]], obuf.at[j], sem)
            for j in range(NB)
        ]
        pltpu.sync_copy(coords_hbm.at[pl.ds(b, NB)], cv)
        for c in copies:
            c.wait()

        for j in range(NB):
            def row_body(l, c2):
                v = obuf[j, l, pl.ds(48, 16)]
                cvec = cv[j, l, pl.ds(0, 16)]
                obuf[j, l, pl.ds(48, 16)] = jnp.where(is_emb, v, cvec)
                return c2
            lax.fori_loop(0, L, row_body, 0)

        pltpu.sync_copy(obuf, out_hbm.at[pl.ds(b, NB)])
        return carry

    lax.fori_loop(0, N_CHUNKS, chunk_body, 0)


def kernel(x, coordinates, table):
    tpad = jnp.pad(table, ((0, 0), (0, 4)))
    cpad = jnp.pad(coordinates, ((0, 0), (0, 0), (12, 0)))
    return _emb_kernel(x.astype(jnp.int32), cpad, tpad)


# speedup vs baseline: 41.4915x; 1.9499x over previous
import functools

import jax
import jax.numpy as jnp
from jax import lax
from jax.experimental import pallas as pl
from jax.experimental.pallas import tpu as pltpu
from jax.experimental.pallas import tpu_sc as plsc

B = 4096
L = 50
D_O = 64
VOCAB = 1000000
NC, NS = 2, 16
NW = NC * NS
B_PER_W = B // NW
NB = 8
N_CHUNKS = B_PER_W // NB

_mesh = plsc.VectorSubcoreMesh(core_axis_name="c", subcore_axis_name="s")


@functools.partial(
    pl.kernel,
    mesh=_mesh,
    out_type=jax.ShapeDtypeStruct((B, L, D_O), jnp.float32),
    scratch_types=[
        pltpu.VMEM((NB, L), jnp.int32),
        pltpu.VMEM((NB, L, 16), jnp.float32),
        pltpu.VMEM((NB, L, D_O), jnp.float32),
        pltpu.SemaphoreType.DMA,
    ],
    compiler_params=pltpu.CompilerParams(use_tc_tiling_on_sc=False),
)
def _emb_kernel(x_hbm, coords_hbm, table_hbm, out_hbm, xv, cv, obuf, sem):
    wid = lax.axis_index("s") * NC + lax.axis_index("c")
    b0 = wid * B_PER_W

    lane = lax.iota(jnp.int32, 16)
    is_emb = lane < 12

    def chunk_body(i, carry):
        b = b0 + i * NB
        pltpu.sync_copy(x_hbm.at[pl.ds(b, NB)], xv)
        copies = [
            pltpu.async_copy(table_hbm.at[xv.at[j]], obuf.at[j], sem)
            for j in range(NB)
        ]
        pltpu.sync_copy(coords_hbm.at[pl.ds(b, NB)], cv)
        for c in copies:
            c.wait()

        for j in range(NB):
            def row_body(l, c2):
                v = obuf[j, l, pl.ds(48, 16)]
                cvec = cv[j, l, pl.ds(0, 16)]
                obuf[j, l, pl.ds(48, 16)] = jnp.where(is_emb, v, cvec)
                return c2
            lax.fori_loop(0, L, row_body, 0)

        pltpu.sync_copy(obuf, out_hbm.at[pl.ds(b, NB)])
        return carry

    lax.fori_loop(0, N_CHUNKS, chunk_body, 0)


_TR = 12800
_NBLK = (VOCAB + _TR - 1) // _TR
_VPAD = _NBLK * _TR


def _tp_body(t_ref, o_ref):
    blk = t_ref[...]
    rows = jnp.transpose(blk, (1, 0))
    lo = jnp.pad(rows[: _TR // 2], ((0, 0), (0, 4)))
    hi = jnp.pad(rows[_TR // 2 :], ((0, 0), (0, 4)))
    o_ref[...] = jnp.concatenate([lo, hi], axis=1)


def _transpose_pad(table_t):
    return pl.pallas_call(
        _tp_body,
        grid=(_NBLK,),
        in_specs=[pl.BlockSpec((60, _TR), lambda i: (0, i))],
        out_specs=pl.BlockSpec((_TR // 2, 2 * D_O), lambda i: (i, 0)),
        out_shape=jax.ShapeDtypeStruct((_VPAD // 2, 2 * D_O), jnp.float32),
    )(table_t)


def kernel(x, coordinates, table):
    tpad = _transpose_pad(table.T).reshape(_VPAD, D_O)
    cpad = jnp.pad(coordinates, ((0, 0), (0, 0), (12, 0)))
    xi = x.astype(jnp.int32)
    blk = xi // _TR
    w = xi - blk * _TR
    p = (w >= _TR // 2).astype(jnp.int32)
    xf = blk * _TR + 2 * (w - (_TR // 2) * p) + p
    return _emb_kernel(xf, cpad, tpad)


# speedup vs baseline: 54.7140x; 1.3187x over previous
import functools

import jax
import jax.numpy as jnp
from jax import lax
from jax.experimental import pallas as pl
from jax.experimental.pallas import tpu as pltpu
from jax.experimental.pallas import tpu_sc as plsc

B = 4096
L = 50
D_O = 64
VOCAB = 1000000
NC, NS = 2, 16
NW = NC * NS
B_PER_W = B // NW
NB = 8
N_CHUNKS = B_PER_W // NB

_mesh = plsc.VectorSubcoreMesh(core_axis_name="c", subcore_axis_name="s")


@functools.partial(
    pl.kernel,
    mesh=_mesh,
    out_type=jax.ShapeDtypeStruct((B, L, D_O), jnp.float32),
    scratch_types=[
        pltpu.VMEM((NB, L), jnp.int32),
        pltpu.VMEM((NB, 16 * L), jnp.float32),
        pltpu.VMEM((NB, L, D_O), jnp.float32),
        pltpu.SemaphoreType.DMA,
    ],
    compiler_params=pltpu.CompilerParams(use_tc_tiling_on_sc=False),
)
def _emb_kernel(x_hbm, coords_hbm, table_hbm, out_hbm, xv, cv, obuf, sem):
    wid = lax.axis_index("s") * NC + lax.axis_index("c")
    b0 = wid * B_PER_W

    lane = lax.iota(jnp.int32, 16)
    is_emb = lane < 12

    def chunk_body(i, carry):
        b = b0 + i * NB
        pltpu.sync_copy(x_hbm.at[pl.ds(b, NB)], xv)
        copies = [
            pltpu.async_copy(table_hbm.at[xv.at[j]], obuf.at[j], sem)
            for j in range(NB)
        ]
        pltpu.sync_copy(coords_hbm.at[pl.ds(b, NB)], cv)
        for c in copies:
            c.wait()

        for j in range(NB):
            def row_body(l, c2):
                v = obuf[j, l, pl.ds(48, 16)]
                cvec = cv[j, pl.ds(16 * l, 16)]
                obuf[j, l, pl.ds(48, 16)] = jnp.where(is_emb, v, cvec)
                return c2
            lax.fori_loop(0, L, row_body, 0)

        pltpu.sync_copy(obuf, out_hbm.at[pl.ds(b, NB)])
        return carry

    lax.fori_loop(0, N_CHUNKS, chunk_body, 0)


_TR = 12800
_NBLK = (VOCAB + _TR - 1) // _TR
_VPAD = _NBLK * _TR


def _tp_body(t_ref, o_ref):
    blk = t_ref[...]
    rows = jnp.transpose(blk, (1, 0))
    lo = jnp.pad(rows[: _TR // 2], ((0, 0), (0, 4)))
    hi = jnp.pad(rows[_TR // 2 :], ((0, 0), (0, 4)))
    o_ref[...] = jnp.concatenate([lo, hi], axis=1)


def _transpose_pad(table_t):
    return pl.pallas_call(
        _tp_body,
        grid=(_NBLK,),
        in_specs=[pl.BlockSpec((60, _TR), lambda i: (0, i))],
        out_specs=pl.BlockSpec((_TR // 2, 2 * D_O), lambda i: (i, 0)),
        out_shape=jax.ShapeDtypeStruct((_VPAD // 2, 2 * D_O), jnp.float32),
    )(table_t)


def kernel(x, coordinates, table):
    tpad = _transpose_pad(table.T).reshape(_VPAD, D_O)
    cpad = jnp.pad(coordinates, ((0, 0), (0, 0), (12, 0))).reshape(B, 16 * L)
    xi = x.astype(jnp.int32)
    blk = xi // _TR
    w = xi - blk * _TR
    p = (w >= _TR // 2).astype(jnp.int32)
    xf = blk * _TR + 2 * (w - (_TR // 2) * p) + p
    return _emb_kernel(xf, cpad, tpad)


# speedup vs baseline: 56.8387x; 1.0388x over previous
import functools

import jax
import jax.numpy as jnp
from jax import lax
from jax.experimental import pallas as pl
from jax.experimental.pallas import tpu as pltpu
from jax.experimental.pallas import tpu_sc as plsc

B = 4096
L = 50
D_O = 64
VOCAB = 1000000
NC, NS = 2, 16
NW = NC * NS
B_PER_W = B // NW
NB = 8
N_CHUNKS = B_PER_W // NB

_mesh = plsc.VectorSubcoreMesh(core_axis_name="c", subcore_axis_name="s")


@functools.partial(
    pl.kernel,
    mesh=_mesh,
    out_type=jax.ShapeDtypeStruct((B, L, D_O), jnp.float32),
    scratch_types=[
        pltpu.VMEM((NB, L), jnp.int32),
        pltpu.VMEM((NB, 16 * L), jnp.float32),
        pltpu.VMEM((NB, L, D_O), jnp.float32),
        pltpu.SemaphoreType.DMA,
    ],
    compiler_params=pltpu.CompilerParams(use_tc_tiling_on_sc=False),
)
def _emb_kernel(x_hbm, coords_hbm, table_hbm, out_hbm, xv, cv, obuf, sem):
    wid = lax.axis_index("s") * NC + lax.axis_index("c")
    b0 = wid * B_PER_W

    lane = lax.iota(jnp.int32, 16)
    is_emb = lane < 12

    def chunk_body(i, carry):
        b = b0 + i * NB
        pltpu.sync_copy(x_hbm.at[pl.ds(b, NB)], xv)
        copies = [
            pltpu.async_copy(table_hbm.at[xv.at[j]], obuf.at[j], sem)
            for j in range(NB)
        ]
        pltpu.sync_copy(coords_hbm.at[pl.ds(b, NB)], cv)
        for c in copies:
            c.wait()

        for j in range(NB):
            def row_body(l, c2):
                v = obuf[j, l, pl.ds(48, 16)]
                cvec = cv[j, pl.ds(16 * l, 16)]
                obuf[j, l, pl.ds(48, 16)] = jnp.where(is_emb, v, cvec)
                return c2
            lax.fori_loop(0, L, row_body, 0)

        pltpu.sync_copy(obuf, out_hbm.at[pl.ds(b, NB)])
        return carry

    lax.fori_loop(0, N_CHUNKS, chunk_body, 0)


_TR = 25600
_NBLK = (VOCAB + _TR - 1) // _TR
_VPAD = _NBLK * _TR


def _tp_body(t_ref, o_ref):
    blk = t_ref[...]
    rows = jnp.transpose(blk, (1, 0))
    lo = jnp.pad(rows[: _TR // 2], ((0, 0), (0, 4)))
    hi = jnp.pad(rows[_TR // 2 :], ((0, 0), (0, 4)))
    o_ref[...] = jnp.concatenate([lo, hi], axis=1)


def _transpose_pad(table_t):
    return pl.pallas_call(
        _tp_body,
        grid=(_NBLK,),
        in_specs=[pl.BlockSpec((60, _TR), lambda i: (0, i))],
        out_specs=pl.BlockSpec((_TR // 2, 2 * D_O), lambda i: (i, 0)),
        out_shape=jax.ShapeDtypeStruct((_VPAD // 2, 2 * D_O), jnp.float32),
    )(table_t)


def kernel(x, coordinates, table):
    tpad = _transpose_pad(table.T).reshape(_VPAD, D_O)
    cpad = jnp.pad(coordinates, ((0, 0), (0, 0), (12, 0))).reshape(B, 16 * L)
    xi = x.astype(jnp.int32)
    blk = xi // _TR
    w = xi - blk * _TR
    p = (w >= _TR // 2).astype(jnp.int32)
    xf = blk * _TR + 2 * (w - (_TR // 2) * p) + p
    return _emb_kernel(xf, cpad, tpad)
